# baseline (speedup 1.0000x reference)
import functools

import jax
import jax.numpy as jnp
from jax import lax
from jax.experimental import pallas as pl
from jax.experimental.pallas import tpu as pltpu
from jax.experimental.pallas import tpu_sc as plsc

EMBED_DIM = 32
HIDDEN_DIM = 64
BATCH = 16384

NUM_CORES = 2
NUM_SUBCORES = 16
NUM_WORKERS = NUM_CORES * NUM_SUBCORES
B_PER_W = BATCH // NUM_WORKERS
LANES = 16
N_VECS = B_PER_W // LANES

_sc_mesh = plsc.VectorSubcoreMesh(core_axis_name="c", subcore_axis_name="s")


def _gather_rows(idx_hbm, emb_hbm, out_hbm, idx_v, sem, base):
    pltpu.sync_copy(idx_hbm.at[pl.ds(base, B_PER_W)], idx_v)
    lane_iota = lax.iota(jnp.int32, LANES)

    @pl.loop(0, N_VECS)
    def _(k):
        v = idx_v[pl.ds(k * LANES, LANES)]
        for j in range(LANES):
            row = jnp.sum(jnp.where(lane_iota == j, v, 0))
            pltpu.async_copy(
                emb_hbm.at[row], out_hbm.at[base + k * LANES + j], sem)
    pltpu.make_async_copy(
        emb_hbm.at[pl.ds(0, B_PER_W)],
        out_hbm.at[pl.ds(base, B_PER_W)], sem).wait()


@functools.partial(
    pl.kernel,
    mesh=_sc_mesh,
    out_type=(
        jax.ShapeDtypeStruct((BATCH, EMBED_DIM), jnp.float32),
        jax.ShapeDtypeStruct((BATCH, EMBED_DIM), jnp.float32),
    ),
    scratch_types=[
        pltpu.VMEM((B_PER_W,), jnp.int32),
        pltpu.VMEM((B_PER_W,), jnp.int32),
        pltpu.SemaphoreType.DMA,
        pltpu.SemaphoreType.DMA,
    ],
    compiler_params=pltpu.CompilerParams(needs_layout_passes=False),
)
def _gather_pairs(from_hbm, to_hbm, emb_hbm, out_f, out_t,
                  idx_f, idx_t, sem_f, sem_t):
    wid = lax.axis_index("s") * NUM_CORES + lax.axis_index("c")
    base = wid * B_PER_W
    _gather_rows(from_hbm, emb_hbm, out_f, idx_f, sem_f, base)
    _gather_rows(to_hbm, emb_hbm, out_t, idx_t, sem_t, base)


def _mlp_body(f_ref, t_ref, w1a_ref, w1b_ref, b1_ref, w2_ref, b2_ref, out_ref):
    h = jnp.dot(f_ref[...], w1a_ref[...], preferred_element_type=jnp.float32)
    h = h + jnp.dot(t_ref[...], w1b_ref[...], preferred_element_type=jnp.float32)
    h = jnp.maximum(h + b1_ref[...], 0.0)
    logit = jnp.sum(h * w2_ref[...], axis=1, keepdims=True) + b2_ref[...]
    out_ref[...] = jax.nn.sigmoid(logit)


def _mlp(f_rows, t_rows, w1a, w1b, b1, w2, b2, block_m=2048):
    grid = (BATCH // block_m,)
    return pl.pallas_call(
        _mlp_body,
        grid=grid,
        in_specs=[
            pl.BlockSpec((block_m, EMBED_DIM), lambda i: (i, 0)),
            pl.BlockSpec((block_m, EMBED_DIM), lambda i: (i, 0)),
            pl.BlockSpec((EMBED_DIM, HIDDEN_DIM), lambda i: (0, 0)),
            pl.BlockSpec((EMBED_DIM, HIDDEN_DIM), lambda i: (0, 0)),
            pl.BlockSpec((1, HIDDEN_DIM), lambda i: (0, 0)),
            pl.BlockSpec((1, HIDDEN_DIM), lambda i: (0, 0)),
            pl.BlockSpec((1, 1), lambda i: (0, 0)),
        ],
        out_specs=pl.BlockSpec((block_m, 1), lambda i: (i, 0)),
        out_shape=jax.ShapeDtypeStruct((BATCH, 1), jnp.float32),
    )(f_rows, t_rows, w1a, w1b, b1, w2, b2)


def kernel(from_node, to_node, emb, fc1_w, fc1_b, fc2_w, fc2_b):
    f_rows, t_rows = _gather_pairs(
        from_node.astype(jnp.int32), to_node.astype(jnp.int32), emb)
    w1a = fc1_w[:, :EMBED_DIM].T
    w1b = fc1_w[:, EMBED_DIM:].T
    b1 = fc1_b.reshape(1, HIDDEN_DIM)
    w2 = fc2_w.reshape(1, HIDDEN_DIM)
    b2 = fc2_b.reshape(1, 1)
    return _mlp(f_rows, t_rows, w1a, w1b, b1, w2, b2)


# speedup vs baseline: 1.8618x; 1.8618x over previous
import functools

import jax
import jax.numpy as jnp
from jax import lax
from jax.experimental import pallas as pl
from jax.experimental.pallas import tpu as pltpu
from jax.experimental.pallas import tpu_sc as plsc

VOCAB = 1_000_000
EMBED_DIM = 32
HIDDEN_DIM = 64
BATCH = 16384

PACK = 128 // EMBED_DIM
G_ROWS = VOCAB // PACK

NUM_CORES = 2
NUM_SUBCORES = 16
NUM_WORKERS = NUM_CORES * NUM_SUBCORES
B_PER_W = BATCH // NUM_WORKERS
CHUNK = 128
N_CHUNKS = B_PER_W // CHUNK

_sc_mesh = plsc.VectorSubcoreMesh(core_axis_name="c", subcore_axis_name="s")


_RB = 4096


def _repack_body(inT_ref, out_ref):
    t = inT_ref[...].T
    t3 = t.reshape(_RB // PACK, PACK, EMBED_DIM)
    out_ref[...] = jnp.concatenate(
        [t3[:, k, :] for k in range(PACK)], axis=1)


def _repack(embT):
    grid = ((VOCAB + _RB - 1) // _RB,)
    return pl.pallas_call(
        _repack_body,
        grid=grid,
        in_specs=[pl.BlockSpec((EMBED_DIM, _RB), lambda i: (0, i))],
        out_specs=pl.BlockSpec((_RB // PACK, 128), lambda i: (i, 0)),
        out_shape=jax.ShapeDtypeStruct((G_ROWS, 128), jnp.float32),
    )(embT)


@functools.partial(
    pl.kernel,
    mesh=_sc_mesh,
    out_type=(
        jax.ShapeDtypeStruct((BATCH, EMBED_DIM), jnp.float32),
        jax.ShapeDtypeStruct((BATCH, EMBED_DIM), jnp.float32),
    ),
    scratch_types=[
        pltpu.VMEM((B_PER_W,), jnp.int32),
        pltpu.VMEM((B_PER_W,), jnp.int32),
        pltpu.VMEM((B_PER_W, EMBED_DIM), jnp.float32),
        pltpu.VMEM((B_PER_W, EMBED_DIM), jnp.float32),
        pltpu.SemaphoreType.DMA,
    ],
    compiler_params=pltpu.CompilerParams(use_tc_tiling_on_sc=False),
)
def _gather_pairs(from_hbm, to_hbm, emb_hbm, out_f, out_t,
                  idx_f, idx_t, rows_f, rows_t, sem):
    wid = lax.axis_index("s") * NUM_CORES + lax.axis_index("c")
    base = wid * B_PER_W
    pltpu.sync_copy(from_hbm.at[pl.ds(base, B_PER_W)], idx_f)
    pltpu.sync_copy(to_hbm.at[pl.ds(base, B_PER_W)], idx_t)
    copies = []
    for j in range(N_CHUNKS):
        s = pl.ds(j * CHUNK, CHUNK)
        copies.append(pltpu.async_copy(emb_hbm.at[idx_f.at[s]], rows_f.at[s], sem))
        copies.append(pltpu.async_copy(emb_hbm.at[idx_t.at[s]], rows_t.at[s], sem))
    for c in copies:
        c.wait()
    pltpu.sync_copy(rows_f, out_f.at[pl.ds(base, B_PER_W)])
    pltpu.sync_copy(rows_t, out_t.at[pl.ds(base, B_PER_W)])


def _mlp_body(f_ref, t_ref, w1a_ref, w1b_ref, b1_ref, w2_ref, b2_ref, out_ref):
    h = jnp.dot(f_ref[...], w1a_ref[...], preferred_element_type=jnp.float32)
    h = h + jnp.dot(t_ref[...], w1b_ref[...], preferred_element_type=jnp.float32)
    h = jnp.maximum(h + b1_ref[...], 0.0)
    logit = jnp.sum(h * w2_ref[...], axis=1, keepdims=True) + b2_ref[...]
    out_ref[...] = jax.nn.sigmoid(logit)


def _mlp(f_rows, t_rows, w1a, w1b, b1, w2, b2, block_m=2048):
    grid = (BATCH // block_m,)
    return pl.pallas_call(
        _mlp_body,
        grid=grid,
        in_specs=[
            pl.BlockSpec((block_m, EMBED_DIM), lambda i: (i, 0)),
            pl.BlockSpec((block_m, EMBED_DIM), lambda i: (i, 0)),
            pl.BlockSpec((EMBED_DIM, HIDDEN_DIM), lambda i: (0, 0)),
            pl.BlockSpec((EMBED_DIM, HIDDEN_DIM), lambda i: (0, 0)),
            pl.BlockSpec((1, HIDDEN_DIM), lambda i: (0, 0)),
            pl.BlockSpec((1, HIDDEN_DIM), lambda i: (0, 0)),
            pl.BlockSpec((1, 1), lambda i: (0, 0)),
        ],
        out_specs=pl.BlockSpec((block_m, 1), lambda i: (i, 0)),
        out_shape=jax.ShapeDtypeStruct((BATCH, 1), jnp.float32),
    )(f_rows, t_rows, w1a, w1b, b1, w2, b2)


def kernel(from_node, to_node, emb, fc1_w, fc1_b, fc2_w, fc2_b):
    g = _repack(emb.T)
    emb_rm = jnp.reshape(g, (VOCAB, EMBED_DIM))
    f_rows, t_rows = _gather_pairs(
        from_node.astype(jnp.int32), to_node.astype(jnp.int32), emb_rm)
    w1a = fc1_w[:, :EMBED_DIM].T
    w1b = fc1_w[:, EMBED_DIM:].T
    b1 = fc1_b.reshape(1, HIDDEN_DIM)
    w2 = fc2_w.reshape(1, HIDDEN_DIM)
    b2 = fc2_b.reshape(1, 1)
    return _mlp(f_rows, t_rows, w1a, w1b, b1, w2, b2)


# speedup vs baseline: 3.2462x; 1.7436x over previous
import functools

import jax
import jax.numpy as jnp
from jax import lax
from jax.experimental import pallas as pl
from jax.experimental.pallas import tpu as pltpu
from jax.experimental.pallas import tpu_sc as plsc

VOCAB = 1_000_000
EMBED_DIM = 32
HIDDEN_DIM = 64
BATCH = 16384

GROUP = 512
N_GROUPS = (VOCAB + GROUP - 1) // GROUP
G_ROWS = N_GROUPS * 128

NUM_CORES = 2
NUM_SUBCORES = 16
NUM_WORKERS = NUM_CORES * NUM_SUBCORES
B_PER_W = BATCH // NUM_WORKERS
LANES = 16
N_VECS = B_PER_W // LANES
CHUNK = 128
N_CHUNKS = B_PER_W // CHUNK

_sc_mesh = plsc.VectorSubcoreMesh(core_axis_name="c", subcore_axis_name="s")


_RB = 4096
_GPB = _RB // GROUP


def _repack_body(inT_ref, out_ref):
    x = inT_ref[...]
    for g in range(_GPB):
        sub = x[:, g * GROUP:(g + 1) * GROUP]
        m = sub.reshape(EMBED_DIM, 4, 128)
        m = m.transpose(1, 0, 2).reshape(128, 128)
        out_ref[g * 128:(g + 1) * 128, :] = m.T


def _repack(embT):
    grid = ((VOCAB + _RB - 1) // _RB,)
    return pl.pallas_call(
        _repack_body,
        grid=grid,
        in_specs=[pl.BlockSpec((EMBED_DIM, _RB), lambda i: (0, i))],
        out_specs=pl.BlockSpec((_GPB * 128, 128), lambda i: (i, 0)),
        out_shape=jax.ShapeDtypeStruct((G_ROWS, 128), jnp.float32),
    )(embT)


@functools.partial(
    pl.kernel,
    mesh=_sc_mesh,
    out_type=(
        jax.ShapeDtypeStruct((EMBED_DIM, BATCH), jnp.float32),
        jax.ShapeDtypeStruct((EMBED_DIM, BATCH), jnp.float32),
    ),
    scratch_types=[
        pltpu.VMEM((B_PER_W,), jnp.int32),
        pltpu.VMEM((B_PER_W,), jnp.int32),
        pltpu.VMEM((B_PER_W,), jnp.int32),
        pltpu.VMEM((B_PER_W, 128), jnp.float32),
        pltpu.VMEM((EMBED_DIM, B_PER_W), jnp.float32),
        pltpu.SemaphoreType.DMA,
    ],
    compiler_params=pltpu.CompilerParams(needs_layout_passes=False),
)
def _gather_pairs(from_hbm, to_hbm, g_hbm, out_f, out_t,
                  idx_v, row_v, seg_v, rows_v, ft_v, sem):
    wid = lax.axis_index("s") * NUM_CORES + lax.axis_index("c")
    base = wid * B_PER_W

    for idx_hbm, outT in ((from_hbm, out_f), (to_hbm, out_t)):
        pltpu.sync_copy(idx_hbm.at[pl.ds(base, B_PER_W)], idx_v)

        @pl.loop(0, N_VECS)
        def _(i):
            s = pl.ds(i * LANES, LANES)
            v = idx_v[s]
            row_v[s] = (v >> 9) * 128 + (v & 127)
            seg_v[s] = ((v >> 7) & 3) * EMBED_DIM

        for c in range(N_CHUNKS):
            s = pl.ds(c * CHUNK, CHUNK)
            pltpu.async_copy(g_hbm.at[row_v.at[s]], rows_v.at[s], sem)
        pltpu.make_async_copy(
            g_hbm.at[pl.ds(0, B_PER_W)], rows_v, sem).wait()

        lane_i = lax.iota(jnp.int32, LANES)

        @pl.loop(0, N_VECS)
        def _(i):
            s = pl.ds(i * LANES, LANES)
            rr = lane_i + i * LANES
            seg = seg_v[s]
            for d in range(EMBED_DIM):
                ft_v[d, s] = plsc.load_gather(rows_v, [rr, seg + d])

        pltpu.sync_copy(ft_v, outT.at[:, pl.ds(base, B_PER_W)])


def _mlp_body(f_ref, t_ref, w1a_ref, w1b_ref, b1_ref, w2_ref, b2_ref, out_ref):
    h = jnp.dot(w1a_ref[...], f_ref[...], preferred_element_type=jnp.float32)
    h = h + jnp.dot(w1b_ref[...], t_ref[...], preferred_element_type=jnp.float32)
    h = jnp.maximum(h + b1_ref[...], 0.0)
    logit = jnp.dot(w2_ref[...], h, preferred_element_type=jnp.float32)
    out_ref[...] = jax.nn.sigmoid(logit + b2_ref[...])


def _mlp(fT, tT, w1a, w1b, b1, w2, b2, block_n=2048):
    grid = (BATCH // block_n,)
    return pl.pallas_call(
        _mlp_body,
        grid=grid,
        in_specs=[
            pl.BlockSpec((EMBED_DIM, block_n), lambda i: (0, i)),
            pl.BlockSpec((EMBED_DIM, block_n), lambda i: (0, i)),
            pl.BlockSpec((HIDDEN_DIM, EMBED_DIM), lambda i: (0, 0)),
            pl.BlockSpec((HIDDEN_DIM, EMBED_DIM), lambda i: (0, 0)),
            pl.BlockSpec((HIDDEN_DIM, 1), lambda i: (0, 0)),
            pl.BlockSpec((1, HIDDEN_DIM), lambda i: (0, 0)),
            pl.BlockSpec((1, 1), lambda i: (0, 0)),
        ],
        out_specs=pl.BlockSpec((1, block_n), lambda i: (0, i)),
        out_shape=jax.ShapeDtypeStruct((1, BATCH), jnp.float32),
    )(fT, tT, w1a, w1b, b1, w2, b2)


def kernel(from_node, to_node, emb, fc1_w, fc1_b, fc2_w, fc2_b):
    g = _repack(emb.T)
    fT, tT = _gather_pairs(
        from_node.astype(jnp.int32), to_node.astype(jnp.int32), g)
    w1a = fc1_w[:, :EMBED_DIM]
    w1b = fc1_w[:, EMBED_DIM:]
    b1 = fc1_b.reshape(HIDDEN_DIM, 1)
    w2 = fc2_w.reshape(1, HIDDEN_DIM)
    b2 = fc2_b.reshape(1, 1)
    out = _mlp(fT, tT, w1a, w1b, b1, w2, b2)
    return out.reshape(BATCH, 1)


# speedup vs baseline: 3.3651x; 1.0366x over previous
import functools

import jax
import jax.numpy as jnp
from jax import lax
from jax.experimental import pallas as pl
from jax.experimental.pallas import tpu as pltpu
from jax.experimental.pallas import tpu_sc as plsc

VOCAB = 1_000_000
EMBED_DIM = 32
HIDDEN_DIM = 64
BATCH = 16384

GROUP = 512
N_GROUPS = (VOCAB + GROUP - 1) // GROUP
G_ROWS = N_GROUPS * 128

NUM_CORES = 2
NUM_SUBCORES = 16
NUM_WORKERS = NUM_CORES * NUM_SUBCORES
B_PER_W = BATCH // NUM_WORKERS
LANES = 16
N_VECS = B_PER_W // LANES
CHUNK = 128
N_CHUNKS = B_PER_W // CHUNK

_sc_mesh = plsc.VectorSubcoreMesh(core_axis_name="c", subcore_axis_name="s")


_RB = 4096
_GPB = _RB // GROUP


def _repack_body(inT_ref, out_ref):
    x = inT_ref[...]
    for g in range(_GPB):
        sub = x[:, g * GROUP:(g + 1) * GROUP]
        m = sub.reshape(EMBED_DIM, 4, 128)
        m = m.transpose(1, 0, 2).reshape(128, 128)
        out_ref[g * 128:(g + 1) * 128, :] = m.T


def _repack(embT):
    grid = ((VOCAB + _RB - 1) // _RB,)
    return pl.pallas_call(
        _repack_body,
        grid=grid,
        in_specs=[pl.BlockSpec((EMBED_DIM, _RB), lambda i: (0, i))],
        out_specs=pl.BlockSpec((_GPB * 128, 128), lambda i: (i, 0)),
        out_shape=jax.ShapeDtypeStruct((G_ROWS, 128), jnp.float32),
    )(embT)


@functools.partial(
    pl.kernel,
    mesh=_sc_mesh,
    out_type=(
        jax.ShapeDtypeStruct((BATCH, EMBED_DIM), jnp.float32),
        jax.ShapeDtypeStruct((BATCH, EMBED_DIM), jnp.float32),
    ),
    scratch_types=[
        pltpu.VMEM((B_PER_W,), jnp.int32),
        pltpu.VMEM((B_PER_W,), jnp.int32),
        pltpu.VMEM((B_PER_W,), jnp.int32),
        pltpu.VMEM((B_PER_W,), jnp.int32),
        pltpu.VMEM((B_PER_W, EMBED_DIM), jnp.float32),
        pltpu.VMEM((B_PER_W, EMBED_DIM), jnp.float32),
        pltpu.SemaphoreType.DMA,
    ],
    compiler_params=pltpu.CompilerParams(use_tc_tiling_on_sc=False),
)
def _gather_pairs(from_hbm, to_hbm, g_hbm, out_f, out_t,
                  idx_f, idx_t, q_f, q_t, rows_f, rows_t, sem):
    wid = lax.axis_index("s") * NUM_CORES + lax.axis_index("c")
    base = wid * B_PER_W
    pltpu.sync_copy(from_hbm.at[pl.ds(base, B_PER_W)], idx_f)
    pltpu.sync_copy(to_hbm.at[pl.ds(base, B_PER_W)], idx_t)

    copies = []
    for idx_v, q_v, rows_v in ((idx_f, q_f, rows_f), (idx_t, q_t, rows_t)):
        @pl.loop(0, N_VECS)
        def _(i):
            s = pl.ds(i * LANES, LANES)
            v = idx_v[s]
            q_v[s] = (v >> 9) * 512 + (v & 127) * 4 + ((v >> 7) & 3)

        for c in range(N_CHUNKS):
            s = pl.ds(c * CHUNK, CHUNK)
            copies.append(
                pltpu.async_copy(g_hbm.at[q_v.at[s]], rows_v.at[s], sem))
    for c in copies:
        c.wait()
    pltpu.sync_copy(rows_f, out_f.at[pl.ds(base, B_PER_W)])
    pltpu.sync_copy(rows_t, out_t.at[pl.ds(base, B_PER_W)])


def _mlp_body(f_ref, t_ref, w1a_ref, w1b_ref, b1_ref, w2_ref, b2_ref, out_ref):
    h = jnp.dot(f_ref[...], w1a_ref[...], preferred_element_type=jnp.float32)
    h = h + jnp.dot(t_ref[...], w1b_ref[...], preferred_element_type=jnp.float32)
    h = jnp.maximum(h + b1_ref[...], 0.0)
    logit = jnp.sum(h * w2_ref[...], axis=1, keepdims=True) + b2_ref[...]
    out_ref[...] = jax.nn.sigmoid(logit)


def _mlp(f_rows, t_rows, w1a, w1b, b1, w2, b2, block_m=2048):
    grid = (BATCH // block_m,)
    return pl.pallas_call(
        _mlp_body,
        grid=grid,
        in_specs=[
            pl.BlockSpec((block_m, EMBED_DIM), lambda i: (i, 0)),
            pl.BlockSpec((block_m, EMBED_DIM), lambda i: (i, 0)),
            pl.BlockSpec((EMBED_DIM, HIDDEN_DIM), lambda i: (0, 0)),
            pl.BlockSpec((EMBED_DIM, HIDDEN_DIM), lambda i: (0, 0)),
            pl.BlockSpec((1, HIDDEN_DIM), lambda i: (0, 0)),
            pl.BlockSpec((1, HIDDEN_DIM), lambda i: (0, 0)),
            pl.BlockSpec((1, 1), lambda i: (0, 0)),
        ],
        out_specs=pl.BlockSpec((block_m, 1), lambda i: (i, 0)),
        out_shape=jax.ShapeDtypeStruct((BATCH, 1), jnp.float32),
    )(f_rows, t_rows, w1a, w1b, b1, w2, b2)


def kernel(from_node, to_node, emb, fc1_w, fc1_b, fc2_w, fc2_b):
    g = _repack(emb.T)
    g_rm = jnp.reshape(g, (G_ROWS * 4, EMBED_DIM))
    f_rows, t_rows = _gather_pairs(
        from_node.astype(jnp.int32), to_node.astype(jnp.int32), g_rm)
    w1a = fc1_w[:, :EMBED_DIM].T
    w1b = fc1_w[:, EMBED_DIM:].T
    b1 = fc1_b.reshape(1, HIDDEN_DIM)
    w2 = fc2_w.reshape(1, HIDDEN_DIM)
    b2 = fc2_b.reshape(1, 1)
    return _mlp(f_rows, t_rows, w1a, w1b, b1, w2, b2)
